# baseline (speedup 1.0000x reference)
import functools

import jax
import jax.numpy as jnp
from jax import lax
from jax.experimental import pallas as pl
from jax.experimental.pallas import tpu as pltpu
from jax.experimental.pallas import tpu_sc as plsc

BZ = 256
C = 128
V = 256
E = 2048
NW = 32
NC = 4
CB = BZ // NC
BPW = CB // NW

_mesh = plsc.VectorSubcoreMesh(core_axis_name="c", subcore_axis_name="s")


def _make_build_a(base):
    @functools.partial(
        pl.kernel,
        mesh=_mesh,
        out_type=jax.ShapeDtypeStruct((CB, V, V), jnp.float32),
        scratch_types=[
            pltpu.VMEM((2, E), jnp.int32),
            pltpu.VMEM((V, V), jnp.float32),
        ],
        compiler_params=pltpu.CompilerParams(needs_layout_passes=False),
    )
    def _build_a(edge_hbm, a_hbm, idx_v, a_v):
        wid = lax.axis_index("s") * 2 + lax.axis_index("c")

        zv = jnp.zeros((16,), jnp.float32)

        def zero_body(i, carry):
            for j in range(16):
                a_v[i, pl.ds(j * 16, 16)] = zv
            return carry

        lax.fori_loop(0, V, zero_body, 0)

        def scatter_pass(val):
            vals = jnp.full((16,), val, jnp.float32)

            def edge_body(i, carry):
                lv = idx_v[0, pl.ds(i * 16, 16)]
                rv = idx_v[1, pl.ds(i * 16, 16)]
                plsc.addupdate_scatter(a_v, [lv, rv], vals)
                return carry

            lax.fori_loop(0, E // 16, edge_body, 0)

        def batch_body(k, carry):
            b = wid * BPW + k
            pltpu.sync_copy(edge_hbm.at[base + b], idx_v)
            scatter_pass(1.0)
            pltpu.sync_copy(a_v, a_hbm.at[b])
            scatter_pass(-1.0)
            return carry

        lax.fori_loop(0, BPW, batch_body, 0)

    return _build_a


_build_a_chunks = [_make_build_a(c * CB) for c in range(NC)]


TB = 16


def _tc_first(a_ref, x_ref, w_ref, b_ref, o_ref):
    w = w_ref[...].astype(jnp.bfloat16)
    x = x_ref[...].astype(jnp.bfloat16)
    y = lax.dot_general(w, x, (((1,), (1,)), ((), ())),
                        preferred_element_type=jnp.float32)
    y = (y + b_ref[...].reshape(C, 1, 1)).astype(jnp.bfloat16)
    a = a_ref[...].astype(jnp.bfloat16)
    o_ref[...] = lax.dot_general(
        a, y, (((2,), (2,)), ((0,), (1,))),
        preferred_element_type=jnp.float32)


def _tc_rest(a_ref, x_ref, w_ref, b_ref, oin_ref, o_ref):
    del oin_ref
    _tc_first(a_ref, x_ref, w_ref, b_ref, o_ref)


def _make_tc(c, first):
    common = dict(
        grid=(CB // TB,),
        out_specs=pl.BlockSpec((TB, V, C), lambda i, c=c: (c * CB // TB + i, 0, 0)),
        out_shape=jax.ShapeDtypeStruct((BZ, V, C), jnp.float32),
        compiler_params=pltpu.CompilerParams(
            dimension_semantics=("arbitrary",)),
    )
    in_specs = [
        pl.BlockSpec((TB, V, V), lambda i: (i, 0, 0)),
        pl.BlockSpec((TB, C, V), lambda i, c=c: (c * CB // TB + i, 0, 0)),
        pl.BlockSpec((C, C), lambda i: (0, 0)),
        pl.BlockSpec((C, 1), lambda i: (0, 0)),
    ]
    if first:
        return pl.pallas_call(_tc_first, in_specs=in_specs, **common)
    in_specs.append(pl.BlockSpec(memory_space=pltpu.MemorySpace.HBM))
    return pl.pallas_call(_tc_rest, in_specs=in_specs,
                          input_output_aliases={4: 0}, **common)


_tc_chunks = [_make_tc(c, first=(c == 0)) for c in range(NC)]


def kernel(x, edge_index, W, b):
    edge_index = edge_index.astype(jnp.int32)
    b2 = b.reshape(C, 1)
    a0 = _build_a_chunks[0](edge_index)
    out = _tc_chunks[0](a0, x, W, b2)
    for c in range(1, NC):
        a_c = _build_a_chunks[c](edge_index)
        out = _tc_chunks[c](a_c, x, W, b2, out)
    return out


# speedup vs baseline: 1.4700x; 1.4700x over previous
import functools

import jax
import jax.numpy as jnp
from jax import lax
from jax.experimental import pallas as pl
from jax.experimental.pallas import tpu as pltpu
from jax.experimental.pallas import tpu_sc as plsc

BZ = 256
C = 128
V = 256
E = 2048
NW = 32
NC = 4
CB = BZ // NC
BPW = CB // NW

_mesh = plsc.VectorSubcoreMesh(core_axis_name="c", subcore_axis_name="s")


def _make_build_a(base):
    @functools.partial(
        pl.kernel,
        mesh=_mesh,
        out_type=jax.ShapeDtypeStruct((CB, V, V), jnp.float32),
        scratch_types=[
            pltpu.VMEM((2, E), jnp.int32),
            pltpu.VMEM((V, V), jnp.float32),
        ],
        compiler_params=pltpu.CompilerParams(needs_layout_passes=False),
    )
    def _build_a(edge_hbm, a_hbm, idx_v, a_v):
        wid = lax.axis_index("s") * 2 + lax.axis_index("c")

        zv = jnp.zeros((16,), jnp.float32)

        def zero_body(i, carry):
            for j in range(16):
                a_v[i, pl.ds(j * 16, 16)] = zv
            return carry

        lax.fori_loop(0, V, zero_body, 0)

        def scatter_pass(val):
            vals = jnp.full((16,), val, jnp.float32)

            def edge_body(i, carry):
                lv = idx_v[0, pl.ds(i * 16, 16)]
                rv = idx_v[1, pl.ds(i * 16, 16)]
                plsc.addupdate_scatter(a_v, [lv, rv], vals)
                return carry

            lax.fori_loop(0, E // 16, edge_body, 0)

        def batch_body(k, carry):
            b = wid * BPW + k
            pltpu.sync_copy(edge_hbm.at[base + b], idx_v)
            scatter_pass(1.0)
            pltpu.sync_copy(a_v, a_hbm.at[b])
            scatter_pass(-1.0)
            return carry

        lax.fori_loop(0, BPW, batch_body, 0)

    return _build_a


_build_a_chunks = [_make_build_a(c * CB) for c in range(NC)]


TB = 16


def _tc_first(a_ref, x_ref, w_ref, b_ref, o_ref):
    w = w_ref[...].astype(jnp.bfloat16)
    bias = b_ref[...]
    for j in range(TB):
        y = jnp.dot(w, x_ref[j].astype(jnp.bfloat16),
                    preferred_element_type=jnp.float32) + bias
        o_ref[j] = jnp.dot(a_ref[j].astype(jnp.bfloat16),
                           y.astype(jnp.bfloat16).T,
                           preferred_element_type=jnp.float32)


def _tc_rest(a_ref, x_ref, w_ref, b_ref, oin_ref, o_ref):
    del oin_ref
    _tc_first(a_ref, x_ref, w_ref, b_ref, o_ref)


def _make_tc(c, first):
    common = dict(
        grid=(CB // TB,),
        out_specs=pl.BlockSpec((TB, V, C), lambda i, c=c: (c * CB // TB + i, 0, 0)),
        out_shape=jax.ShapeDtypeStruct((BZ, V, C), jnp.float32),
        compiler_params=pltpu.CompilerParams(
            dimension_semantics=("parallel",)),
    )
    in_specs = [
        pl.BlockSpec((TB, V, V), lambda i: (i, 0, 0)),
        pl.BlockSpec((TB, C, V), lambda i, c=c: (c * CB // TB + i, 0, 0)),
        pl.BlockSpec((C, C), lambda i: (0, 0)),
        pl.BlockSpec((C, 1), lambda i: (0, 0)),
    ]
    if first:
        return pl.pallas_call(_tc_first, in_specs=in_specs, **common)
    in_specs.append(pl.BlockSpec(memory_space=pltpu.MemorySpace.HBM))
    return pl.pallas_call(_tc_rest, in_specs=in_specs,
                          input_output_aliases={4: 0}, **common)


_tc_chunks = [_make_tc(c, first=(c == 0)) for c in range(NC)]


def kernel(x, edge_index, W, b):
    edge_index = edge_index.astype(jnp.int32)
    b2 = b.reshape(C, 1)
    a0 = _build_a_chunks[0](edge_index)
    out = _tc_chunks[0](a0, x, W, b2)
    for c in range(1, NC):
        a_c = _build_a_chunks[c](edge_index)
        out = _tc_chunks[c](a_c, x, W, b2, out)
    return out
